# baseline (speedup 1.0000x reference)
import functools
import math

import jax
import jax.numpy as jnp
from jax import lax
from jax.experimental import pallas as pl
from jax.experimental.pallas import tpu as pltpu
from jax.experimental.pallas import tpu_sc as plsc


_N = 64
_D = 32
_H = 32
_K = 8
_TB = 16
_M = _TB * _N


def _gelu(x):
    return 0.5 * x * (1.0 + jax.lax.erf(x * (1.0 / math.sqrt(2.0))))


_C10 = (((1,), (0,)), ((), ()))
_C11 = (((1,), (1,)), ((), ()))
_BMM = (((2,), (2,)), ((0,), (0,)))


def _token_kernel(state_ref, tf_ref, sp_ref, tproj_ref, qw_ref, kw_ref,
                  qb_ref, kb_ref, wxi_ref, wxj_ref,
                  spA_ref, spC_ref, w2_ref, b2_ref, w3_ref,
                  wc0_ref, wc1_ref, wc2_ref, wc3_ref, wc4_ref,
                  mbias_ref, alpha_ref, r_ref,
                  lr_ref, vals_ref, idx_ref):
    f32 = jnp.float32
    N = _N
    K = _K
    M = _M
    qb = qb_ref[...]
    kb = kb_ref[...]
    wxi = wxi_ref[...]
    wxj = wxj_ref[...]
    w2 = w2_ref[...]
    b2 = b2_ref[...]
    w3 = w3_ref[...]
    alpha = alpha_ref[...]
    r_row = r_ref[...]

    i0 = jax.lax.broadcasted_iota(jnp.int32, (N, N), 0)
    i1 = jax.lax.broadcasted_iota(jnp.int32, (N, N), 1)
    cumU = jnp.where(i1 <= i0, 1.0, 0.0).astype(f32)
    jjf = jax.lax.broadcasted_iota(jnp.int32, (M, N), 1).astype(f32)
    slot_i = jax.lax.broadcasted_iota(jnp.int32, (M, K, N), 1)

    def tile_tok(x):
        a, b = x.shape
        return jnp.broadcast_to(x.reshape(1, a, b), (_TB, a, b)).reshape(M, b)

    spA_t = tile_tok(spA_ref[...])
    spC_t = tile_tok(spC_ref[...])
    sp3 = sp_ref[...].reshape(1, N, _D)
    wc0 = wc0_ref[...].reshape(1, N, N)
    wc1 = wc1_ref[...].reshape(1, N, N)
    wc2 = wc2_ref[...].reshape(1, N, N)
    wc3 = wc3_ref[...].reshape(1, N, N)
    wc4 = wc4_ref[...].reshape(1, N, N)
    mbias = mbias_ref[...].reshape(1, N, N)

    state_blk = state_ref[...]
    t0 = jax.lax.broadcasted_iota(jnp.int32, (_TB, _TB), 0)
    t1 = jax.lax.broadcasted_iota(jnp.int32, (_TB, _TB), 1)
    eyeT = jnp.where(t0 == t1, 1.0, 0.0).astype(f32)
    sT = jax.lax.dot_general(state_blk, eyeT, (((0,), (0,)), ((), ())),
                             preferred_element_type=f32)
    xi_b = jnp.concatenate(
        [jnp.broadcast_to(sT[:, t:t + 1], (N, N)) for t in range(_TB)],
        axis=0)
    s_col = xi_b[:, 0:1]
    xj3 = jnp.broadcast_to(state_blk.reshape(_TB, 1, N),
                           (_TB, N, N))
    xi3 = xi_b.reshape(_TB, N, N)

    tf = (tf_ref[...] + sp3).reshape(M, _D)
    proj = jax.lax.dot_general(tf, tproj_ref[...], _C10,
                               preferred_element_type=f32)
    feats = jnp.concatenate([s_col, proj], axis=1)
    q = jax.lax.dot_general(feats, qw_ref[...], _C10,
                            preferred_element_type=f32) + qb
    k = jax.lax.dot_general(feats, kw_ref[...], _C10,
                            preferred_element_type=f32) + kb
    scores = jax.lax.dot_general(q.reshape(_TB, N, _D), k.reshape(_TB, N, _D),
                                 _BMM, preferred_element_type=f32)
    scores = (scores / (_D ** 0.5)).reshape(M, N)

    cur = scores
    m_list, ix_list = [], []
    for _ in range(K):
        m = jnp.max(cur, axis=1, keepdims=True)
        idxm = jnp.where(cur == m, jjf, float(N))
        minidx = jnp.min(idxm, axis=1, keepdims=True)
        cur = jnp.where(idxm == minidx, -jnp.inf, cur)
        m_list.append(m)
        ix_list.append(minidx)
    keep = cur == -jnp.inf
    keepf = jnp.where(keep, 1.0, 0.0).astype(f32)

    kcum = jax.lax.dot_general(keepf, cumU, _C11,
                               preferred_element_type=f32)
    slotv = (kcum - 0.5).astype(jnp.int32).reshape(M, 1, N)
    keep3 = keepf.reshape(M, 1, N)
    S3 = jnp.where((jnp.broadcast_to(slotv, (M, K, N)) == slot_i) &
                   (jnp.broadcast_to(keep3, (M, K, N)) > 0.0),
                   1.0, 0.0).astype(f32)

    A = s_col * wxi + spA_t
    C = s_col * wxj + spC_t
    C_sel = jax.lax.dot_general(S3.reshape(_TB, N * K, N),
                                C.reshape(_TB, N, _H),
                                (((2,), (1,)), ((0,), (0,))),
                                preferred_element_type=f32)
    A_sel = jnp.broadcast_to(A.reshape(M, 1, _H), (M, K, _H))
    h1 = _gelu(A_sel.reshape(M * K, _H) + C_sel.reshape(M * K, _H))
    h2 = _gelu(jax.lax.dot_general(h1, w2, _C10,
                                   preferred_element_type=f32) + b2)
    f4s = jnp.sum(h2.reshape(M, K, _H) * w3.reshape(1, 1, _H),
                  axis=2)

    f4d = jnp.sum(S3 * f4s.reshape(M, K, 1), axis=1)

    alpha3 = alpha.reshape(1, 1, N)
    holl = xj3 / (1.0 + alpha3 * xj3)
    msgs = (wc0 * xj3 + wc1 * xi3 * xj3 + wc2 * holl +
            wc3 * xi3 * holl + wc4 * f4d.reshape(_TB, N, N) +
            mbias)

    rowmax = m_list[0]
    e = jnp.where(keep, jnp.exp(scores - rowmax), 0.0)
    z = jnp.sum(e, axis=1, keepdims=True)
    attn = (e / z).reshape(_TB, N, N)

    agg = jnp.sum(attn * msgs, axis=2)
    lr_ref[...] = r_row + agg
    vals_ref[...] = jnp.concatenate(
        [jnp.exp(mm - rowmax) for mm in m_list], axis=1) / z
    idx_ref[...] = jnp.concatenate(ix_list, axis=1).astype(jnp.int32)


def _rep(shape):
    nd = len(shape)
    return pl.BlockSpec(shape, lambda i, _nd=nd: (0,) * _nd)


def kernel(state, temporal_feat, species_emb, q_W, q_b, k_W, k_b, tproj_W,
           form_coefs, form_gates_raw, holling_alpha_raw,
           mlp_W1, mlp_b1, mlp_W2, mlp_b2, mlp_W3, mlp_b3, r):
    B, T, N = state.shape
    D = species_emb.shape[1]
    H = mlp_W2.shape[0]
    BT = B * T

    gates = jax.nn.sigmoid(form_gates_raw)
    wc = form_coefs * gates
    alpha = (jax.nn.softplus(holling_alpha_raw) + 0.01).reshape(1, N)
    spA = species_emb @ mlp_W1[2:2 + D] + mlp_b1
    spC = species_emb @ mlp_W1[2 + D:2 + 2 * D]
    wxi = mlp_W1[0].reshape(1, H)
    wxj = mlp_W1[1].reshape(1, H)
    mbias = wc[4] * mlp_b3[0]
    w3 = mlp_W3.reshape(1, H)

    state2 = state.reshape(BT, N)
    tf2 = temporal_feat.reshape(BT, N, D)

    grid = (BT // _TB,)
    out_shape = (
        jax.ShapeDtypeStruct((BT, N), jnp.float32),
        jax.ShapeDtypeStruct((BT * N, _K), jnp.float32),
        jax.ShapeDtypeStruct((BT * N, _K), jnp.int32),
    )
    in_specs = [
        pl.BlockSpec((_TB, N), lambda i: (i, 0)),
        pl.BlockSpec((_TB, N, D), lambda i: (i, 0, 0)),
        _rep((N, D)),
        _rep((D, D)),
        _rep((1 + D, D)),
        _rep((1 + D, D)),
        _rep((1, D)),
        _rep((1, D)),
        _rep((1, H)),
        _rep((1, H)),
        _rep((N, H)),
        _rep((N, H)),
        _rep((H, H)),
        _rep((1, H)),
        _rep((1, H)),
        _rep((N, N)),
        _rep((N, N)),
        _rep((N, N)),
        _rep((N, N)),
        _rep((N, N)),
        _rep((N, N)),
        _rep((1, N)),
        _rep((1, N)),
    ]
    out_specs = (
        pl.BlockSpec((_TB, N), lambda i: (i, 0)),
        pl.BlockSpec((_M, _K), lambda i: (i, 0)),
        pl.BlockSpec((_M, _K), lambda i: (i, 0)),
    )

    lr2, vals2, idx2 = pl.pallas_call(
        _token_kernel,
        grid=grid,
        in_specs=in_specs,
        out_specs=out_specs,
        out_shape=out_shape,
    )(state2, tf2, species_emb, tproj_W, q_W, k_W,
      q_b.reshape(1, D), k_b.reshape(1, D), wxi, wxj, spA, spC,
      mlp_W2, mlp_b2.reshape(1, H), w3,
      wc[0], wc[1], wc[2], wc[3], wc[4], mbias, alpha, r.reshape(1, N))

    attn_flat = _sc_scatter(vals2.reshape(-1), idx2.reshape(-1), BT * N, N)
    return lr2.reshape(B, T, N), attn_flat.reshape(B, T, N, N)


_NW = 32
_CH = 64


def _sc_scatter(vals_flat, idx_flat, rows, n):
    rpw = rows // _NW
    nch = rpw // _CH

    @functools.partial(
        pl.kernel,
        mesh=plsc.VectorSubcoreMesh(core_axis_name="c", subcore_axis_name="s"),
        out_type=jax.ShapeDtypeStruct((rows * n,), jnp.float32),
        scratch_types=[
            pltpu.VMEM((_CH * _K,), jnp.float32),
            pltpu.VMEM((_CH * _K,), jnp.int32),
            pltpu.VMEM((_CH * _K // 128, 128), jnp.int32),
            pltpu.VMEM((_CH * n,), jnp.float32),
            pltpu.SemaphoreType.DMA,
        ],
    )
    def body(vals_hbm, idx_hbm, out_hbm, vals_v, idx_v, fi_v, zb, sem):
        wid = lax.axis_index("s") * 2 + lax.axis_index("c")
        zero = jnp.zeros((16,), jnp.float32)
        for i in range(_CH * n // 16):
            zb[pl.ds(i * 16, 16)] = zero
        lane = lax.iota(jnp.int32, 16)
        rowoff = lax.shift_right_logical(lane, 3)
        base = wid * rpw
        for c in range(nch):
            pltpu.sync_copy(zb, out_hbm.at[pl.ds((base + c * _CH) * n,
                                                 _CH * n)])
        for c in range(nch):
            rowbase = base + c * _CH
            pltpu.sync_copy(vals_hbm.at[pl.ds(rowbase * _K, _CH * _K)],
                            vals_v)
            pltpu.sync_copy(idx_hbm.at[pl.ds(rowbase * _K, _CH * _K)],
                            idx_v)
            for s in range(_CH * _K // 16):
                iv = idx_v[pl.ds(s * 16, 16)]
                fi = (rowbase + rowoff + (s * 2)) * n + iv
                fi_v[s // 8, pl.ds((s % 8) * 16, 16)] = fi
            for g in range(_CH * _K // 128):
                pltpu.async_copy(vals_v.at[pl.ds(g * 128, 128)],
                                 out_hbm.at[fi_v.at[g]], sem).wait()

    return body(vals_flat, idx_flat)


# speedup vs baseline: 1.0097x; 1.0097x over previous
import functools
import math

import jax
import jax.numpy as jnp
from jax import lax
from jax.experimental import pallas as pl
from jax.experimental.pallas import tpu as pltpu
from jax.experimental.pallas import tpu_sc as plsc


_N = 64
_D = 32
_H = 32
_K = 8
_TB = 16
_M = _TB * _N


def _gelu(x):
    return 0.5 * x * (1.0 + jax.lax.erf(x * (1.0 / math.sqrt(2.0))))


_C10 = (((1,), (0,)), ((), ()))
_C11 = (((1,), (1,)), ((), ()))
_BMM = (((2,), (2,)), ((0,), (0,)))


def _token_kernel(state_ref, tf_ref, sp_ref, tproj_ref, qw_ref, kw_ref,
                  qb_ref, kb_ref, wxi_ref, wxj_ref,
                  spA_ref, spC_ref, w2_ref, b2_ref, w3_ref,
                  wc0_ref, wc1_ref, wc2_ref, wc3_ref, wc4_ref,
                  mbias_ref, alpha_ref, r_ref,
                  lr_ref, vals_ref, idx_ref):
    f32 = jnp.float32
    N = _N
    K = _K
    M = _M
    qb = qb_ref[...]
    kb = kb_ref[...]
    wxi = wxi_ref[...]
    wxj = wxj_ref[...]
    w2 = w2_ref[...]
    b2 = b2_ref[...]
    w3 = w3_ref[...]
    alpha = alpha_ref[...]
    r_row = r_ref[...]

    i0 = jax.lax.broadcasted_iota(jnp.int32, (N, N), 0)
    i1 = jax.lax.broadcasted_iota(jnp.int32, (N, N), 1)
    cumU = jnp.where(i1 <= i0, 1.0, 0.0).astype(f32)
    jjf = jax.lax.broadcasted_iota(jnp.int32, (M, N), 1).astype(f32)
    slot_i = jax.lax.broadcasted_iota(jnp.int32, (M, K, N), 1)

    def tile_tok(x):
        a, b = x.shape
        return jnp.broadcast_to(x.reshape(1, a, b), (_TB, a, b)).reshape(M, b)

    spA_t = tile_tok(spA_ref[...])
    spC_t = tile_tok(spC_ref[...])
    sp3 = sp_ref[...].reshape(1, N, _D)
    wc0 = wc0_ref[...].reshape(1, N, N)
    wc1 = wc1_ref[...].reshape(1, N, N)
    wc2 = wc2_ref[...].reshape(1, N, N)
    wc3 = wc3_ref[...].reshape(1, N, N)
    wc4 = wc4_ref[...].reshape(1, N, N)
    mbias = mbias_ref[...].reshape(1, N, N)

    state_blk = state_ref[...]
    t0 = jax.lax.broadcasted_iota(jnp.int32, (_TB, _TB), 0)
    t1 = jax.lax.broadcasted_iota(jnp.int32, (_TB, _TB), 1)
    eyeT = jnp.where(t0 == t1, 1.0, 0.0).astype(f32)
    sT = jax.lax.dot_general(state_blk, eyeT, (((0,), (0,)), ((), ())),
                             preferred_element_type=f32)
    xi_b = jnp.concatenate(
        [jnp.broadcast_to(sT[:, t:t + 1], (N, N)) for t in range(_TB)],
        axis=0)
    s_col = xi_b[:, 0:1]
    xj3 = jnp.broadcast_to(state_blk.reshape(_TB, 1, N),
                           (_TB, N, N))
    xi3 = xi_b.reshape(_TB, N, N)

    tf = (tf_ref[...] + sp3).reshape(M, _D)
    proj = jax.lax.dot_general(tf, tproj_ref[...], _C10,
                               preferred_element_type=f32)
    feats = jnp.concatenate([s_col, proj], axis=1)
    q = jax.lax.dot_general(feats, qw_ref[...], _C10,
                            preferred_element_type=f32) + qb
    k = jax.lax.dot_general(feats, kw_ref[...], _C10,
                            preferred_element_type=f32) + kb
    scores = jax.lax.dot_general(q.reshape(_TB, N, _D), k.reshape(_TB, N, _D),
                                 _BMM, preferred_element_type=f32)
    scores = (scores / (_D ** 0.5)).reshape(M, N)

    cur = scores
    m_list, ix_list = [], []
    for _ in range(K):
        m = jnp.max(cur, axis=1, keepdims=True)
        idxm = jnp.where(cur == m, jjf, float(N))
        minidx = jnp.min(idxm, axis=1, keepdims=True)
        cur = jnp.where(idxm == minidx, -jnp.inf, cur)
        m_list.append(m)
        ix_list.append(minidx)
    keep = cur == -jnp.inf
    keepf = jnp.where(keep, 1.0, 0.0).astype(f32)

    kcum = jax.lax.dot_general(keepf, cumU, _C11,
                               preferred_element_type=f32)
    slotv = (kcum - 0.5).astype(jnp.int32).reshape(M, 1, N)
    keep3 = keepf.reshape(M, 1, N)
    S3 = jnp.where((jnp.broadcast_to(slotv, (M, K, N)) == slot_i) &
                   (jnp.broadcast_to(keep3, (M, K, N)) > 0.0),
                   1.0, 0.0).astype(f32)

    A = s_col * wxi + spA_t
    C = s_col * wxj + spC_t
    C_sel = jax.lax.dot_general(S3.reshape(_TB, N * K, N),
                                C.reshape(_TB, N, _H),
                                (((2,), (1,)), ((0,), (0,))),
                                preferred_element_type=f32)
    A_sel = jnp.broadcast_to(A.reshape(M, 1, _H), (M, K, _H))
    h1 = _gelu(A_sel.reshape(M * K, _H) + C_sel.reshape(M * K, _H))
    h2 = _gelu(jax.lax.dot_general(h1, w2, _C10,
                                   preferred_element_type=f32) + b2)
    f4s = jnp.sum(h2.reshape(M, K, _H) * w3.reshape(1, 1, _H),
                  axis=2)

    f4d = jnp.sum(S3 * f4s.reshape(M, K, 1), axis=1)

    alpha3 = alpha.reshape(1, 1, N)
    holl = xj3 / (1.0 + alpha3 * xj3)
    msgs = (wc0 * xj3 + wc1 * xi3 * xj3 + wc2 * holl +
            wc3 * xi3 * holl + wc4 * f4d.reshape(_TB, N, N) +
            mbias)

    rowmax = m_list[0]
    e = jnp.where(keep, jnp.exp(scores - rowmax), 0.0)
    z = jnp.sum(e, axis=1, keepdims=True)
    attn = (e / z).reshape(_TB, N, N)

    agg = jnp.sum(attn * msgs, axis=2)
    lr_ref[...] = r_row + agg
    vals_ref[...] = jnp.concatenate(
        [jnp.exp(mm - rowmax) for mm in m_list], axis=1) / z
    idx_ref[...] = jnp.concatenate(ix_list, axis=1).astype(jnp.int32)


def _rep(shape):
    nd = len(shape)
    return pl.BlockSpec(shape, lambda i, _nd=nd: (0,) * _nd)


def kernel(state, temporal_feat, species_emb, q_W, q_b, k_W, k_b, tproj_W,
           form_coefs, form_gates_raw, holling_alpha_raw,
           mlp_W1, mlp_b1, mlp_W2, mlp_b2, mlp_W3, mlp_b3, r):
    B, T, N = state.shape
    D = species_emb.shape[1]
    H = mlp_W2.shape[0]
    BT = B * T

    gates = jax.nn.sigmoid(form_gates_raw)
    wc = form_coefs * gates
    alpha = (jax.nn.softplus(holling_alpha_raw) + 0.01).reshape(1, N)
    spA = species_emb @ mlp_W1[2:2 + D] + mlp_b1
    spC = species_emb @ mlp_W1[2 + D:2 + 2 * D]
    wxi = mlp_W1[0].reshape(1, H)
    wxj = mlp_W1[1].reshape(1, H)
    mbias = wc[4] * mlp_b3[0]
    w3 = mlp_W3.reshape(1, H)

    state2 = state.reshape(BT, N)
    tf2 = temporal_feat.reshape(BT, N, D)

    grid = (BT // _TB,)
    out_shape = (
        jax.ShapeDtypeStruct((BT, N), jnp.float32),
        jax.ShapeDtypeStruct((BT * N, _K), jnp.float32),
        jax.ShapeDtypeStruct((BT * N, _K), jnp.int32),
    )
    in_specs = [
        pl.BlockSpec((_TB, N), lambda i: (i, 0)),
        pl.BlockSpec((_TB, N, D), lambda i: (i, 0, 0)),
        _rep((N, D)),
        _rep((D, D)),
        _rep((1 + D, D)),
        _rep((1 + D, D)),
        _rep((1, D)),
        _rep((1, D)),
        _rep((1, H)),
        _rep((1, H)),
        _rep((N, H)),
        _rep((N, H)),
        _rep((H, H)),
        _rep((1, H)),
        _rep((1, H)),
        _rep((N, N)),
        _rep((N, N)),
        _rep((N, N)),
        _rep((N, N)),
        _rep((N, N)),
        _rep((N, N)),
        _rep((1, N)),
        _rep((1, N)),
    ]
    out_specs = (
        pl.BlockSpec((_TB, N), lambda i: (i, 0)),
        pl.BlockSpec((_M, _K), lambda i: (i, 0)),
        pl.BlockSpec((_M, _K), lambda i: (i, 0)),
    )

    lr2, vals2, idx2 = pl.pallas_call(
        _token_kernel,
        grid=grid,
        in_specs=in_specs,
        out_specs=out_specs,
        out_shape=out_shape,
    )(state2, tf2, species_emb, tproj_W, q_W, k_W,
      q_b.reshape(1, D), k_b.reshape(1, D), wxi, wxj, spA, spC,
      mlp_W2, mlp_b2.reshape(1, H), w3,
      wc[0], wc[1], wc[2], wc[3], wc[4], mbias, alpha, r.reshape(1, N))

    attn_flat = _sc_scatter(vals2.reshape(-1), idx2.reshape(-1), BT * N, N)
    return lr2.reshape(B, T, N), attn_flat.reshape(B, T, N, N)


_NW = 32
_CH = 64


def _sc_scatter(vals_flat, idx_flat, rows, n):
    rpw = rows // _NW
    nch = rpw // _CH

    @functools.partial(
        pl.kernel,
        mesh=plsc.VectorSubcoreMesh(core_axis_name="c", subcore_axis_name="s"),
        out_type=jax.ShapeDtypeStruct((rows * n,), jnp.float32),
        scratch_types=[
            pltpu.VMEM((rpw * _K,), jnp.float32),
            pltpu.VMEM((rpw * _K,), jnp.int32),
            pltpu.VMEM((rpw * _K // 128, 128), jnp.int32),
            pltpu.VMEM((_CH * n,), jnp.float32),
            pltpu.SemaphoreType.DMA,
        ],
    )
    def body(vals_hbm, idx_hbm, out_hbm, vals_v, idx_v, fi_v, zb, sem):
        wid = lax.axis_index("s") * 2 + lax.axis_index("c")
        zero = jnp.zeros((16,), jnp.float32)
        for i in range(_CH * n // 16):
            zb[pl.ds(i * 16, 16)] = zero
        lane = lax.iota(jnp.int32, 16)
        rowoff = lax.shift_right_logical(lane, 3)
        base = wid * rpw
        zd = [pltpu.async_copy(zb,
                               out_hbm.at[pl.ds((base + c * _CH) * n,
                                                _CH * n)], sem)
              for c in range(nch)]
        pltpu.sync_copy(vals_hbm.at[pl.ds(base * _K, rpw * _K)], vals_v)
        pltpu.sync_copy(idx_hbm.at[pl.ds(base * _K, rpw * _K)], idx_v)
        for s in range(rpw * _K // 16):
            iv = idx_v[pl.ds(s * 16, 16)]
            fi = (base + rowoff + (s * 2)) * n + iv
            fi_v[s // 8, pl.ds((s % 8) * 16, 16)] = fi
        for d in zd:
            d.wait()
        sd = [pltpu.async_copy(vals_v.at[pl.ds(g * 128, 128)],
                               out_hbm.at[fi_v.at[g]], sem)
              for g in range(rpw * _K // 128)]
        for d in sd:
            d.wait()

    return body(vals_flat, idx_flat)
